# baseline (speedup 1.0000x reference)
import functools

import jax
import jax.numpy as jnp
from jax import lax
from jax.experimental import pallas as pl
from jax.experimental.pallas import tpu as pltpu
from jax.experimental.pallas import tpu_sc as plsc

VOCAB = 100000
EMBED = 128
PADC = 16
NC, NS = 2, 16
NW = NC * NS
VB = 8192
PACK = 8
NBUF = 8
SEQP = 208


def _proj_body(emb_ref, w_ref, out_ref):
    h = jnp.dot(emb_ref[...], w_ref[...], preferred_element_type=jnp.float32)
    u = lax.bitcast_convert_type(h, jnp.int32) + 32768
    lo = lax.shift_right_logical(u, 16)
    hi = u & jnp.int32(-65536)
    out_ref[...] = lo | pltpu.roll(hi, EMBED - 1, 1)


def _project(embf, w_big):
    rows = VOCAB // PACK
    rb = VB // PACK
    return pl.pallas_call(
        _proj_body,
        grid=((rows + rb - 1) // rb,),
        in_specs=[
            pl.BlockSpec((rb, PACK * EMBED), lambda i: (i, 0)),
            pl.BlockSpec((PACK * EMBED, EMBED), lambda i: (0, 0)),
        ],
        out_specs=pl.BlockSpec((rb, EMBED), lambda i: (i, 0)),
        out_shape=jax.ShapeDtypeStruct((rows, EMBED), jnp.int32),
    )(embf, w_big)


def _make_sc_pool(batch, seq, cls):
    bpw = batch // NW
    s_a = 96
    s_b = seq - s_a
    ngrp = SEQP // 16
    mesh = plsc.VectorSubcoreMesh(core_axis_name="c", subcore_axis_name="s")

    @functools.partial(
        pl.kernel,
        out_type=jax.ShapeDtypeStruct((batch * PADC,), jnp.float32),
        mesh=mesh,
        compiler_params=pltpu.CompilerParams(use_tc_tiling_on_sc=False),
        scratch_types=(
            [pltpu.VMEM((bpw * seq,), jnp.int32)]
            + [pltpu.VMEM((SEQP,), jnp.int32) for _ in range(NBUF)]
            + [pltpu.VMEM((bpw * PADC,), jnp.float32)]
            + [pltpu.SemaphoreType.DMA for _ in range(NBUF)]
        ),
    )
    def sc_pool(ptp, x_hbm, out_hbm, idx_v, *rest):
        bufs = rest[:NBUF]
        probs_v = rest[NBUF]
        sems = rest[NBUF + 1:]

        cid = lax.axis_index("c")
        sid = lax.axis_index("s")
        wid = sid * NC + cid
        base = wid * bpw

        pltpu.sync_copy(x_hbm.at[pl.ds(base * seq, bpw * seq)], idx_v)

        def fire(b, buf, sem):
            off = b * seq
            pltpu.async_copy(ptp.at[idx_v.at[pl.ds(off, s_a)]],
                             buf.at[pl.ds(0, s_a)], sem)
            pltpu.async_copy(ptp.at[idx_v.at[pl.ds(off + s_a, s_b)]],
                             buf.at[pl.ds(s_a, s_b)], sem)

        def wait_buf(buf, sem):
            pltpu.make_async_copy(ptp.at[pl.ds(0, seq)],
                                  buf.at[pl.ds(0, seq)], sem).wait()

        lane = lax.broadcasted_iota(jnp.int32, (PADC,), 0)
        zero_f = jnp.zeros((PADC,), jnp.float32)
        zero_i = jnp.zeros((PADC,), jnp.int32)
        mask_hi = jnp.full((PADC,), -65536, jnp.int32)

        def vperm(v, p):
            return lax.gather(
                v, p[:, None],
                dimension_numbers=lax.GatherDimensionNumbers(
                    offset_dims=(), collapsed_slice_dims=(0,),
                    start_index_map=(0,)),
                slice_sizes=(1,),
                mode=lax.GatherScatterMode.PROMISE_IN_BOUNDS)

        def lanesum(v):
            for k in (1, 2, 4, 8):
                v = v + vperm(v, lane ^ k)
            return v

        def consume(buf, b):
            a0 = [zero_f] * 2
            a1 = [zero_f] * 2
            for g in range(ngrp):
                v = buf[pl.ds(g * 16, 16)]
                c0 = lax.bitcast_convert_type(v << 16, jnp.float32)
                c1 = lax.bitcast_convert_type(v & mask_hi, jnp.float32)
                a0[g % 2] = a0[g % 2] + c0
                a1[g % 2] = a1[g % 2] + c1
            l0 = lanesum(a0[0] + a0[1])
            l1 = lanesum(a1[0] + a1[1])
            logits = jnp.where(lane == 1, l1, l0)
            e = jnp.exp(logits)
            e_swap = vperm(e, lane ^ 1)
            probs_v[pl.ds(b * PADC, PADC)] = e / (e + e_swap)

        for j in range(NBUF):
            bufs[j][pl.ds(SEQP - 16, 16)] = zero_i
            fire(j, bufs[j], sems[j])

        def body(i, _):
            for j in range(NBUF):
                b = i * NBUF + j
                wait_buf(bufs[j], sems[j])
                consume(bufs[j], b)

                @pl.when(b + NBUF < bpw)
                def _():
                    fire(b + NBUF, bufs[j], sems[j])
            return 0

        lax.fori_loop(0, bpw // NBUF, body, 0)
        pltpu.sync_copy(probs_v, out_hbm.at[pl.ds(base * PADC, bpw * PADC)])

    return sc_pool


def kernel(x, emb_table, W_out):
    batch, seq = x.shape
    cls = W_out.shape[0]
    w2p = jnp.zeros((EMBED, PADC), jnp.float32)
    w2p = w2p.at[:, :cls].set(W_out.T / seq)
    w_big = jnp.einsum('ke,dc->kdec', jnp.eye(PACK, dtype=jnp.float32),
                       w2p).reshape(PACK * EMBED, EMBED)
    embf = emb_table.reshape(VOCAB // PACK, PACK * EMBED)
    ptp = _project(embf, w_big).reshape(-1)
    xf = (x.reshape(-1) * PADC).astype(jnp.int32)
    probs_flat = _make_sc_pool(batch, seq, cls)(ptp, xf)
    return probs_flat.reshape(batch, PADC)[:, :cls]


# speedup vs baseline: 1.3646x; 1.3646x over previous
import functools

import jax
import jax.numpy as jnp
from jax import lax
from jax.experimental import pallas as pl
from jax.experimental.pallas import tpu as pltpu
from jax.experimental.pallas import tpu_sc as plsc

VOCAB = 100000
EMBED = 128
PADC = 16
NC, NS = 2, 16
NW = NC * NS
VB = 8192
PACK = 8
NBUF = 8
SEQP = 208


def _proj_body(emb_ref, w_ref, out_ref):
    h = jnp.dot(emb_ref[:, 0, :], w_ref[0],
                preferred_element_type=jnp.float32)
    for k in range(1, PACK):
        h = h + jnp.dot(emb_ref[:, k, :], w_ref[k],
                        preferred_element_type=jnp.float32)
    u = lax.bitcast_convert_type(h, jnp.int32) + 32768
    lo = lax.shift_right_logical(u, 16)
    hi = u & jnp.int32(-65536)
    out_ref[...] = lo | pltpu.roll(hi, EMBED - 1, 1)


def _project(emb3, w_big3):
    rows = VOCAB // PACK
    rb = VB // PACK
    return pl.pallas_call(
        _proj_body,
        grid=((rows + rb - 1) // rb,),
        in_specs=[
            pl.BlockSpec((rb, PACK, EMBED), lambda i: (i, 0, 0)),
            pl.BlockSpec((PACK, EMBED, EMBED), lambda i: (0, 0, 0)),
        ],
        out_specs=pl.BlockSpec((rb, EMBED), lambda i: (i, 0)),
        out_shape=jax.ShapeDtypeStruct((rows, EMBED), jnp.int32),
    )(emb3, w_big3)


def _make_sc_pool(batch, seq, cls):
    bpw = batch // NW
    s_a = 96
    s_b = seq - s_a
    ngrp = SEQP // 16
    mesh = plsc.VectorSubcoreMesh(core_axis_name="c", subcore_axis_name="s")

    @functools.partial(
        pl.kernel,
        out_type=jax.ShapeDtypeStruct((batch * PADC,), jnp.float32),
        mesh=mesh,
        compiler_params=pltpu.CompilerParams(use_tc_tiling_on_sc=False),
        scratch_types=(
            [pltpu.VMEM((bpw * seq,), jnp.int32)]
            + [pltpu.VMEM((SEQP,), jnp.int32) for _ in range(NBUF)]
            + [pltpu.VMEM((bpw * PADC,), jnp.float32)]
            + [pltpu.SemaphoreType.DMA for _ in range(NBUF)]
        ),
    )
    def sc_pool(ptp, x_hbm, out_hbm, idx_v, *rest):
        bufs = rest[:NBUF]
        probs_v = rest[NBUF]
        sems = rest[NBUF + 1:]

        cid = lax.axis_index("c")
        sid = lax.axis_index("s")
        wid = sid * NC + cid
        base = wid * bpw

        pltpu.sync_copy(x_hbm.at[pl.ds(base * seq, bpw * seq)], idx_v)

        def fire(b, buf, sem):
            off = b * seq
            pltpu.async_copy(ptp.at[idx_v.at[pl.ds(off, s_a)]],
                             buf.at[pl.ds(0, s_a)], sem)
            pltpu.async_copy(ptp.at[idx_v.at[pl.ds(off + s_a, s_b)]],
                             buf.at[pl.ds(s_a, s_b)], sem)

        def wait_buf(buf, sem):
            pltpu.make_async_copy(ptp.at[pl.ds(0, seq)],
                                  buf.at[pl.ds(0, seq)], sem).wait()

        lane = lax.broadcasted_iota(jnp.int32, (PADC,), 0)
        zero_f = jnp.zeros((PADC,), jnp.float32)
        zero_i = jnp.zeros((PADC,), jnp.int32)
        mask_hi = jnp.full((PADC,), -65536, jnp.int32)

        def vperm(v, p):
            return lax.gather(
                v, p[:, None],
                dimension_numbers=lax.GatherDimensionNumbers(
                    offset_dims=(), collapsed_slice_dims=(0,),
                    start_index_map=(0,)),
                slice_sizes=(1,),
                mode=lax.GatherScatterMode.PROMISE_IN_BOUNDS)

        def lanesum(v):
            for k in (1, 2, 4, 8):
                v = v + vperm(v, lane ^ k)
            return v

        def consume(buf, b):
            a0 = [zero_f] * 2
            a1 = [zero_f] * 2
            for g in range(ngrp):
                v = buf[pl.ds(g * 16, 16)]
                c0 = lax.bitcast_convert_type(v << 16, jnp.float32)
                c1 = lax.bitcast_convert_type(v & mask_hi, jnp.float32)
                a0[g % 2] = a0[g % 2] + c0
                a1[g % 2] = a1[g % 2] + c1
            l0 = lanesum(a0[0] + a0[1])
            l1 = lanesum(a1[0] + a1[1])
            logits = jnp.where(lane == 1, l1, l0)
            e = jnp.exp(logits)
            e_swap = vperm(e, lane ^ 1)
            probs_v[pl.ds(b * PADC, PADC)] = e / (e + e_swap)

        for j in range(NBUF):
            bufs[j][pl.ds(SEQP - 16, 16)] = zero_i
            fire(j, bufs[j], sems[j])

        def body(i, _):
            for j in range(NBUF):
                b = i * NBUF + j
                wait_buf(bufs[j], sems[j])
                consume(bufs[j], b)

                @pl.when(b + NBUF < bpw)
                def _():
                    fire(b + NBUF, bufs[j], sems[j])
            return 0

        lax.fori_loop(0, bpw // NBUF, body, 0)
        pltpu.sync_copy(probs_v, out_hbm.at[pl.ds(base * PADC, bpw * PADC)])

    return sc_pool


def kernel(x, emb_table, W_out):
    batch, seq = x.shape
    cls = W_out.shape[0]
    w2p = jnp.zeros((EMBED, PADC), jnp.float32)
    w2p = w2p.at[:, :cls].set(W_out.T / seq)
    w_big3 = jnp.einsum('ke,dc->kdec', jnp.eye(PACK, dtype=jnp.float32),
                        w2p).reshape(PACK, EMBED, EMBED)
    emb3 = emb_table.reshape(VOCAB // PACK, PACK, EMBED)
    ptp = _project(emb3, w_big3).reshape(-1)
    xf = (x.reshape(-1) * PADC).astype(jnp.int32)
    probs_flat = _make_sc_pool(batch, seq, cls)(ptp, xf)
    return probs_flat.reshape(batch, PADC)[:, :cls]
